# baseline (speedup 1.0000x reference)
import functools
import jax
import jax.numpy as jnp
from jax import lax
from jax.experimental import pallas as pl
from jax.experimental.pallas import tpu as pltpu
from jax.experimental.pallas import tpu_sc as plsc

N_NODES = 10000
N_EDGES = 160000
NUM_OBJS = 64
NUM_PREDS = 16
EMB = 64
GDIM = 128
HID = 128
LAT = HID // 8

NP = 10240
EP = 163840
NC, NS = 2, 16
NW = NC * NS
CHUNK = 128
NCHUNK = EP // CHUNK
CPT = NCHUNK // NW
ROWS_PT = NP // NS
BE = 2048
BN = 2048

f32 = jnp.float32
i32 = jnp.int32

EP2 = EP // 2
NCHUNK2 = NCHUNK // 2
CPT2 = NCHUNK2 // NW


def _make_gather_body(h):
  def body(p_hbm, q_hbm, sidx, oidx, g_out, sidx_v, oidx_v,
           bufa0, bufb0, bufw0, bufa1, bufb1, bufw1,
           sema0, semb0, semw0, sema1, semb1, semw1):
    cid = lax.axis_index("c")
    sid = lax.axis_index("s")
    wid = sid * NC + cid
    c0l = wid * CPT2
    pltpu.sync_copy(sidx.at[wid], sidx_v)
    pltpu.sync_copy(oidx.at[wid], oidx_v)

    bufs = ((bufa0, bufb0, bufw0, sema0, semb0, semw0),
            (bufa1, bufb1, bufw1, sema1, semb1, semw1))

    def issue(j, ba, bb, bw, sa, sb, sw):
      pltpu.async_copy(p_hbm.at[sidx_v.at[j]], ba, sa)
      pltpu.async_copy(q_hbm.at[oidx_v.at[j]], bb, sb)

    def wait_write(j, ba, bb, bw, sa, sb, sw):
      pltpu.make_async_copy(bw, g_out.at[pl.ds((c0l + j) * CHUNK, CHUNK)],
                            sw).wait()

    def vadd(ba, bb, bw):
      def rowbody(r, carry):
        for k in range(8):
          sl = pl.ds(k * 16, 16)
          bw[r, sl] = ba[r, sl] + bb[r, sl]
        return carry

      lax.fori_loop(0, CHUNK, rowbody, 0)

    def step(j, has_prev_write, do_issue, ba, bb, bw, sa, sb, sw):
      pltpu.make_async_copy(p_hbm.at[sidx_v.at[j]], ba, sa).wait()
      pltpu.make_async_copy(q_hbm.at[oidx_v.at[j]], bb, sb).wait()
      if has_prev_write:
        wait_write(j - 2, ba, bb, bw, sa, sb, sw)
      vadd(ba, bb, bw)
      if do_issue:
        issue(j + 2, ba, bb, bw, sa, sb, sw)
      pltpu.async_copy(bw, g_out.at[pl.ds((c0l + j) * CHUNK, CHUNK)], sw)

    issue(0, *bufs[0])
    issue(1, *bufs[1])
    for p in range(2):
      step(p, False, True, *bufs[p])

    def body_loop(i, carry):
      for p in range(2):
        j = 2 + i * 2 + p
        step(j, True, True, *bufs[p])
      return carry

    lax.fori_loop(0, (CPT2 - 4) // 2, body_loop, 0)
    for p in range(2):
      step(CPT2 - 2 + p, True, False, *bufs[p])
    for p in range(2):
      wait_write(CPT2 - 2 + p, *bufs[p])

  return body


@functools.cache
def _get_sc_gather(h):
  mesh = plsc.VectorSubcoreMesh(core_axis_name="c", subcore_axis_name="s")
  return pl.kernel(
      _make_gather_body(h),
      out_type=jax.ShapeDtypeStruct((EP2, 128), f32),
      mesh=mesh,
      scratch_types=(
          [pltpu.VMEM((CPT2, CHUNK), i32)] * 2
          + [pltpu.VMEM((CHUNK, 128), f32)] * 6
          + [pltpu.SemaphoreType.DMA] * 6
      ),
      compiler_params=pltpu.CompilerParams(needs_layout_passes=False),
  )


def _sc_gather(h, *args):
  return _get_sc_gather(h)(*args)


SCHUNK = 64
SNCHUNK = EP // SCHUNK
SNCHUNK2 = SNCHUNK // 2
SCPT2 = SNCHUNK2 // NW


def _make_scatter_body(h):
  def body(s_hbm, o_hbm, sidx, oidx, z64, out, sidx_v, oidx_v,
           buf0, buf1, sem0, sem1, pooled):
    cid = lax.axis_index("c")
    sid = lax.axis_index("s")
    pltpu.sync_copy(z64, buf0)
    for k in range(ROWS_PT // SCHUNK):
      pltpu.sync_copy(buf0, pooled.at[pl.ds(sid * ROWS_PT + k * SCHUNK,
                                            SCHUNK)])
    plsc.subcore_barrier()

    c0l = cid * (SNCHUNK2 // NC) + sid * SCPT2
    c0g = h * SNCHUNK2 + c0l
    pltpu.sync_copy(sidx.at[pl.ds(c0g, SCPT2)], sidx_v)
    pltpu.sync_copy(oidx.at[pl.ds(c0g, SCPT2)], oidx_v)

    bufs = ((buf0, sem0), (buf1, sem1))

    def one_pass(val_hbm, idx_v):
      def issue(j, b, s):
        pltpu.async_copy(val_hbm.at[pl.ds((c0l + j) * SCHUNK, SCHUNK)], b, s)

      def drain_add(j, b, s):
        pltpu.make_async_copy(
            val_hbm.at[pl.ds((c0l + j) * SCHUNK, SCHUNK)], b, s).wait()
        pltpu.sync_copy(b, pooled.at[idx_v.at[j]], add=True)

      issue(0, *bufs[0])
      issue(1, *bufs[1])

      def body_loop(i, carry):
        for p in range(2):
          j = i * 2 + p
          drain_add(j, *bufs[p])
          issue(j + 2, *bufs[p])
        return carry

      lax.fori_loop(0, (SCPT2 - 2) // 2, body_loop, 0)
      for p in range(2):
        drain_add(SCPT2 - 2 + p, *bufs[p])

    one_pass(s_hbm, sidx_v)
    one_pass(o_hbm, oidx_v)
    plsc.subcore_barrier()
    pltpu.sync_copy(pooled.at[pl.ds(sid * ROWS_PT, ROWS_PT)],
                    out.at[pl.ds(cid * NP + sid * ROWS_PT, ROWS_PT)])

  return body


@functools.cache
def _get_sc_scatter(h):
  mesh = plsc.VectorSubcoreMesh(core_axis_name="c", subcore_axis_name="s")
  return pl.kernel(
      _make_scatter_body(h),
      out_type=jax.ShapeDtypeStruct((NC * NP, 128), f32),
      mesh=mesh,
      scratch_types=[
          pltpu.VMEM((SCPT2, SCHUNK), i32),
          pltpu.VMEM((SCPT2, SCHUNK), i32),
          pltpu.VMEM((SCHUNK, 128), f32),
          pltpu.VMEM((SCHUNK, 128), f32),
          pltpu.SemaphoreType.DMA,
          pltpu.SemaphoreType.DMA,
          pltpu.VMEM_SHARED((NP, 128), f32),
      ],
      compiler_params=pltpu.CompilerParams(needs_layout_passes=False),
  )


def _sc_scatter(h, *args):
  return _get_sc_scatter(h)(*args)


def _counts_body(sidx, oidx, znp, out, idx_v, counts_v):
  cid = lax.axis_index("c")
  sid = lax.axis_index("s")
  wid = sid * NC + cid
  pltpu.sync_copy(znp, counts_v)
  ones = jnp.full((16,), 1.0, f32)
  c0 = wid * CPT
  for idx_ref in (sidx, oidx):
    pltpu.sync_copy(idx_ref.at[pl.ds(c0, CPT)], idx_v)

    def body(j, carry):
      for k in range(CHUNK // 16):
        idx16 = idx_v[j, pl.ds(k * 16, 16)]
        plsc.addupdate_scatter(counts_v, [idx16], ones)
      return carry

    lax.fori_loop(0, CPT, body, 0)
  pltpu.sync_copy(counts_v, out.at[wid])


@functools.cache
def _get_sc_counts():
  mesh = plsc.VectorSubcoreMesh(core_axis_name="c", subcore_axis_name="s")
  return pl.kernel(
      _counts_body,
      out_type=jax.ShapeDtypeStruct((NW, NP), f32),
      mesh=mesh,
      scratch_types=[
          pltpu.VMEM((CPT, CHUNK), i32),
          pltpu.VMEM((NP,), f32),
      ],
      compiler_params=pltpu.CompilerParams(needs_layout_passes=False),
  )


def _sc_counts(*args):
  return _get_sc_counts()(*args)


def _cmerge_body(c_ref, out_ref):
  s = jnp.sum(c_ref[...], axis=0, keepdims=True)
  out_ref[...] = jnp.maximum(s, 1.0)


def _tc_counts_merge(cparts):
  return pl.pallas_call(
      _cmerge_body,
      grid=(NP // BN,),
      in_specs=[pl.BlockSpec((NW, BN), lambda i: (0, i))],
      out_specs=pl.BlockSpec((1, BN), lambda i: (0, i)),
      out_shape=jax.ShapeDtypeStruct((1, NP), f32),
  )(cparts)


def _node0_body(objs_ref, boxes_ref, tab_ref, w1s_ref, w1o_ref, p_ref, q_ref):
  oh = (objs_ref[...] == lax.broadcasted_iota(i32, (1, NUM_OBJS), 1)
        ).astype(f32)
  emb = jnp.dot(oh, tab_ref[...], preferred_element_type=f32,
                  precision=lax.Precision.HIGHEST)
  tvec = jnp.concatenate([emb, boxes_ref[...]], axis=1)
  p_ref[...] = jnp.dot(tvec, w1s_ref[...], preferred_element_type=f32,
                  precision=lax.Precision.HIGHEST)
  q_ref[...] = jnp.dot(tvec, w1o_ref[...], preferred_element_type=f32,
                  precision=lax.Precision.HIGHEST)


def _tc_node0(objs2d, boxes_p, tab, w1s, w1o):
  return pl.pallas_call(
      _node0_body,
      grid=(NP // BN,),
      in_specs=[
          pl.BlockSpec((BN, 1), lambda i: (i, 0)),
          pl.BlockSpec((BN, 4), lambda i: (i, 0)),
          pl.BlockSpec(tab.shape, lambda i: (0, 0)),
          pl.BlockSpec(w1s.shape, lambda i: (0, 0)),
          pl.BlockSpec(w1o.shape, lambda i: (0, 0)),
      ],
      out_specs=[pl.BlockSpec((BN, 128), lambda i: (i, 0))] * 2,
      out_shape=[jax.ShapeDtypeStruct((NP, 128), f32)] * 2,
  )(objs2d, boxes_p, tab, w1s, w1o)


def _make_edge_call(din_p, dout, onehot_pred, fuse_rel):
  def body(g_ref, pred_ref, *rest):
    if onehot_pred:
      (ptab_ref, w1p_ref, b1a_ref, w1bs_ref, b1bs_ref, w1bp_ref, b1bp_ref,
       w1bo_ref, b1bo_ref, s_ref, p_ref, o_ref) = rest
      pp = jnp.dot(ptab_ref[...], w1p_ref[...], preferred_element_type=f32,
                  precision=lax.Precision.HIGHEST)
      oh = (pred_ref[...] == lax.broadcasted_iota(i32, (1, NUM_PREDS), 1)
            ).astype(f32)
      pcon = jnp.dot(oh, pp, preferred_element_type=f32,
                  precision=lax.Precision.HIGHEST)
    elif fuse_rel:
      (w1p_ref, b1a_ref, w1bs_ref, b1bs_ref, w1bp_ref, b1bp_ref,
       w1bo_ref, b1bo_ref, wr_ref, br_ref, s_ref, p_ref, o_ref) = rest
      pcon = jnp.dot(pred_ref[...], w1p_ref[...], preferred_element_type=f32,
                  precision=lax.Precision.HIGHEST)
    else:
      (w1p_ref, b1a_ref, w1bs_ref, b1bs_ref, w1bp_ref, b1bp_ref,
       w1bo_ref, b1bo_ref, s_ref, p_ref, o_ref) = rest
      pcon = jnp.dot(pred_ref[...], w1p_ref[...], preferred_element_type=f32,
                  precision=lax.Precision.HIGHEST)
    h = jax.nn.relu(g_ref[...] + pcon + b1a_ref[...])
    s_ref[...] = jax.nn.relu(
        jnp.dot(h, w1bs_ref[...], preferred_element_type=f32,
                  precision=lax.Precision.HIGHEST) + b1bs_ref[...])
    newp = jax.nn.relu(
        jnp.dot(h, w1bp_ref[...], preferred_element_type=f32,
                  precision=lax.Precision.HIGHEST) + b1bp_ref[...])
    o_ref[...] = jax.nn.relu(
        jnp.dot(h, w1bo_ref[...], preferred_element_type=f32,
                  precision=lax.Precision.HIGHEST) + b1bo_ref[...])
    if fuse_rel:
      p_ref[...] = (jnp.dot(newp, wr_ref[...], preferred_element_type=f32,
                  precision=lax.Precision.HIGHEST)
                    + br_ref[...])
    else:
      p_ref[...] = newp

  p_width = NUM_PREDS if fuse_rel else dout
  pred_in_w = 1 if onehot_pred else din_p

  def call(g, pred, *weights):
    wspecs = [pl.BlockSpec(w.shape, lambda i: (0, 0)) for w in weights]
    return pl.pallas_call(
        body,
        grid=(EP2 // BE,),
        in_specs=[
            pl.BlockSpec((BE, 128), lambda i: (i, 0)),
            pl.BlockSpec((BE, pred_in_w), lambda i: (i, 0)),
        ] + wspecs,
        out_specs=[
            pl.BlockSpec((BE, 128), lambda i: (i, 0)),
            pl.BlockSpec((BE, p_width), lambda i: (i, 0)),
            pl.BlockSpec((BE, 128), lambda i: (i, 0)),
        ],
        out_shape=[
            jax.ShapeDtypeStruct((EP2, 128), f32),
            jax.ShapeDtypeStruct((EP2, p_width), f32),
            jax.ShapeDtypeStruct((EP2, 128), f32),
        ],
    )(g, pred, *weights)

  return call


def _make_node_call(final):
  if final:
    def body(pa_ref, pb_ref, pc_ref, pd_ref, cnt_ref, w2a_ref, b2a_ref,
             w2b_ref, b2b_ref, wn_ref, bn_ref, a_ref):
      pooled = (pa_ref[...] + pb_ref[...] + pc_ref[...] + pd_ref[...]
                ) / cnt_ref[...]
      h2 = jax.nn.relu(
          jnp.dot(pooled, w2a_ref[...], preferred_element_type=f32,
                  precision=lax.Precision.HIGHEST)
          + b2a_ref[...])
      obj = jax.nn.relu(
          jnp.dot(h2, w2b_ref[...], preferred_element_type=f32,
                  precision=lax.Precision.HIGHEST)
          + b2b_ref[...])
      a_ref[...] = (jnp.dot(obj, wn_ref[...], preferred_element_type=f32,
                  precision=lax.Precision.HIGHEST)
                    + bn_ref[...])
  else:
    def body(pa_ref, pb_ref, pc_ref, pd_ref, cnt_ref, w2a_ref, b2a_ref,
             w2b_ref, b2b_ref, w1s_ref, w1o_ref, a_ref, b_ref):
      pooled = (pa_ref[...] + pb_ref[...] + pc_ref[...] + pd_ref[...]
                ) / cnt_ref[...]
      h2 = jax.nn.relu(
          jnp.dot(pooled, w2a_ref[...], preferred_element_type=f32,
                  precision=lax.Precision.HIGHEST)
          + b2a_ref[...])
      obj = jax.nn.relu(
          jnp.dot(h2, w2b_ref[...], preferred_element_type=f32,
                  precision=lax.Precision.HIGHEST)
          + b2b_ref[...])
      a_ref[...] = jnp.dot(obj, w1s_ref[...], preferred_element_type=f32,
                  precision=lax.Precision.HIGHEST)
      b_ref[...] = jnp.dot(obj, w1o_ref[...], preferred_element_type=f32,
                  precision=lax.Precision.HIGHEST)

  out_w = NUM_OBJS if final else 128
  n_out = 1 if final else 2

  def call(parts_a, parts_b, cnt, *weights):
    wspecs = [pl.BlockSpec(w.shape, lambda i: (0, 0)) for w in weights]
    out_specs = [pl.BlockSpec((BN, out_w), lambda i: (i, 0))] * n_out
    out_shape = [jax.ShapeDtypeStruct((NP, out_w), f32)] * n_out
    return pl.pallas_call(
        body,
        grid=(NP // BN,),
        in_specs=[
            pl.BlockSpec((BN, 128), lambda i: (i, 0)),
            pl.BlockSpec((BN, 128), lambda i: (i + NP // BN, 0)),
            pl.BlockSpec((BN, 128), lambda i: (i, 0)),
            pl.BlockSpec((BN, 128), lambda i: (i + NP // BN, 0)),
            pl.BlockSpec((BN, 1), lambda i: (i, 0)),
        ] + wspecs,
        out_specs=out_specs[0] if final else out_specs,
        out_shape=out_shape[0] if final else out_shape,
    )(parts_a, parts_a, parts_b, parts_b, cnt, *weights)

  return call


_edge_enc0 = _make_edge_call(EMB, GDIM, onehot_pred=True, fuse_rel=False)
_edge_mid = _make_edge_call(GDIM, GDIM, onehot_pred=False, fuse_rel=False)
_edge_enc4 = _make_edge_call(GDIM, LAT, onehot_pred=False, fuse_rel=False)
_edge_dec = _make_edge_call(LAT, EMB, onehot_pred=False, fuse_rel=True)
_node_mid = _make_node_call(final=False)
_node_final = _make_node_call(final=True)


def _split_w1(p, din_o, din_p, dout):
  w1a, w1b, b1b = p['w1a'], p['w1b'], p['b1b']
  return dict(
      w1s=w1a[:din_o], w1p=w1a[din_o:din_o + din_p], w1o=w1a[din_o + din_p:],
      b1a=p['b1a'][None, :],
      w1bs=w1b[:, :HID], b1bs=b1b[None, :HID],
      w1bp=w1b[:, HID:HID + dout], b1bp=b1b[None, HID:HID + dout],
      w1bo=w1b[:, HID + dout:], b1bo=b1b[None, HID + dout:],
      w2a=p['w2a'], b2a=p['b2a'][None, :],
      w2b=p['w2b'], b2b=p['b2b'][None, :],
  )


_LAYERS = [
    ('enc0', EMB + 4, EMB, GDIM),
    ('enc1', GDIM, GDIM, GDIM),
    ('enc2', GDIM, GDIM, GDIM),
    ('enc3', GDIM, GDIM, GDIM),
    ('enc4', GDIM, GDIM, LAT),
    ('dec', LAT, LAT, EMB),
]


def kernel(objs, edges, predicates, boxes, params):
  s_idx = edges[:, 0].astype(i32)
  o_idx = edges[:, 1].astype(i32)
  pad_e = jnp.full((EP - N_EDGES,), N_NODES, i32)
  s_pad = jnp.concatenate([s_idx, pad_e])
  o_pad = jnp.concatenate([o_idx, pad_e])
  sidx2d = s_pad.reshape(NCHUNK, CHUNK)
  oidx2d = o_pad.reshape(NCHUNK, CHUNK)
  sidx3g = s_pad.reshape(2, NW, CPT2, CHUNK)
  oidx3g = o_pad.reshape(2, NW, CPT2, CHUNK)
  sidx64 = s_pad.reshape(SNCHUNK, SCHUNK)
  oidx64 = o_pad.reshape(SNCHUNK, SCHUNK)
  objs2d = jnp.pad(objs.astype(i32), (0, NP - N_NODES)).reshape(NP, 1)
  boxes_p = jnp.pad(boxes, ((0, NP - N_NODES), (0, 0)))
  preds2d = jnp.pad(predicates.astype(i32), (0, EP - N_EDGES)).reshape(EP, 1)
  z64 = jnp.zeros((SCHUNK, 128), f32)
  znp = jnp.zeros((NP,), f32)

  W = {name: _split_w1(params[name], a, b, c) for name, a, b, c in _LAYERS}

  cparts = _sc_counts(sidx2d, oidx2d, znp)
  cnt = _tc_counts_merge(cparts).reshape(NP, 1)

  P, Q = _tc_node0(objs2d, boxes_p, params['obj_table'][:NUM_OBJS],
                   W['enc0']['w1s'], W['enc0']['w1o'])

  preds_h = (preds2d[:EP2], preds2d[EP2:])
  pred_h = [preds_h[0], preds_h[1]]
  rel_h = [None, None]
  logits_full = None
  for li, (name, din_o, din_p, dout) in enumerate(_LAYERS):
    w = W[name]
    common = (w['b1a'], w['w1bs'], w['b1bs'], w['w1bp'], w['b1bp'],
              w['w1bo'], w['b1bo'])
    S_h = [None, None]
    O_h = [None, None]
    parts_h = [None, None]
    for hh in range(2):
      g = _sc_gather(hh, P, Q, sidx3g[hh], oidx3g[hh])
      if name == 'enc0':
        S, newp, O = _edge_enc0(
            g, pred_h[hh], params['pred_table'], w['w1p'], *common)
      elif name == 'enc4':
        S, newp, O = _edge_enc4(g, pred_h[hh], w['w1p'], *common)
      elif name == 'dec':
        S, newp, O = _edge_dec(
            g, pred_h[hh], w['w1p'], *common,
            params['wr'], params['br'][None, :])
        rel_h[hh] = newp
      else:
        S, newp, O = _edge_mid(g, pred_h[hh], w['w1p'], *common)
      if name != 'dec':
        pred_h[hh] = newp
      S_h[hh], O_h[hh] = S, O
    for hh in range(2):
      parts_h[hh] = _sc_scatter(hh, S_h[hh], O_h[hh], sidx64, oidx64, z64)
    if name == 'dec':
      logits_full = _node_final(parts_h[0], parts_h[1], cnt, w['w2a'],
                                w['b2a'], w['w2b'], w['b2b'], params['wn'],
                                params['bn'][None, :])
    else:
      nxt = _LAYERS[li + 1][0]
      P, Q = _node_mid(parts_h[0], parts_h[1], cnt, w['w2a'], w['b2a'],
                       w['w2b'], w['b2b'], W[nxt]['w1s'], W[nxt]['w1o'])

  rel_full = jnp.concatenate(rel_h, axis=0)
  return logits_full[:N_NODES], rel_full[:N_EDGES]


# speedup vs baseline: 1.1730x; 1.1730x over previous
import functools
import jax
import jax.numpy as jnp
from jax import lax
from jax.experimental import pallas as pl
from jax.experimental.pallas import tpu as pltpu
from jax.experimental.pallas import tpu_sc as plsc

N_NODES = 10000
N_EDGES = 160000
NUM_OBJS = 64
NUM_PREDS = 16
EMB = 64
GDIM = 128
HID = 128
LAT = HID // 8

NP = 10240
EP = 163840
NC, NS = 2, 16
NW = NC * NS
CHUNK = 128
NCHUNK = EP // CHUNK
CPT = NCHUNK // NW
ROWS_PT = NP // NS
BE = 2048
BN = 2048

f32 = jnp.float32
i32 = jnp.int32

EP2 = EP // 2
NCHUNK2 = NCHUNK // 2
CPT2 = NCHUNK2 // NW


def _make_gather_body(h):
  def body(p_hbm, q_hbm, sidx, oidx, g_out, sidx_v, oidx_v,
           bufa0, bufb0, bufw0, bufa1, bufb1, bufw1,
           sema0, semb0, semw0, sema1, semb1, semw1):
    cid = lax.axis_index("c")
    sid = lax.axis_index("s")
    wid = sid * NC + cid
    c0l = wid * CPT2
    pltpu.sync_copy(sidx.at[wid], sidx_v)
    pltpu.sync_copy(oidx.at[wid], oidx_v)

    bufs = ((bufa0, bufb0, bufw0, sema0, semb0, semw0),
            (bufa1, bufb1, bufw1, sema1, semb1, semw1))

    def issue(j, ba, bb, bw, sa, sb, sw):
      pltpu.async_copy(p_hbm.at[sidx_v.at[j]], ba, sa)
      pltpu.async_copy(q_hbm.at[oidx_v.at[j]], bb, sb)

    def wait_write(j, ba, bb, bw, sa, sb, sw):
      pltpu.make_async_copy(bw, g_out.at[pl.ds((c0l + j) * CHUNK, CHUNK)],
                            sw).wait()

    def vadd(ba, bb, bw):
      def rowbody(r, carry):
        for k in range(8):
          sl = pl.ds(k * 16, 16)
          bw[r, sl] = ba[r, sl] + bb[r, sl]
        return carry

      lax.fori_loop(0, CHUNK, rowbody, 0)

    def step(j, has_prev_write, do_issue, ba, bb, bw, sa, sb, sw):
      pltpu.make_async_copy(p_hbm.at[sidx_v.at[j]], ba, sa).wait()
      pltpu.make_async_copy(q_hbm.at[oidx_v.at[j]], bb, sb).wait()
      if has_prev_write:
        wait_write(j - 2, ba, bb, bw, sa, sb, sw)
      vadd(ba, bb, bw)
      if do_issue:
        issue(j + 2, ba, bb, bw, sa, sb, sw)
      pltpu.async_copy(bw, g_out.at[pl.ds((c0l + j) * CHUNK, CHUNK)], sw)

    issue(0, *bufs[0])
    issue(1, *bufs[1])
    for p in range(2):
      step(p, False, True, *bufs[p])

    def body_loop(i, carry):
      for p in range(2):
        j = 2 + i * 2 + p
        step(j, True, True, *bufs[p])
      return carry

    lax.fori_loop(0, (CPT2 - 4) // 2, body_loop, 0)
    for p in range(2):
      step(CPT2 - 2 + p, True, False, *bufs[p])
    for p in range(2):
      wait_write(CPT2 - 2 + p, *bufs[p])

  return body


@functools.cache
def _get_sc_gather(h):
  mesh = plsc.VectorSubcoreMesh(core_axis_name="c", subcore_axis_name="s")
  return pl.kernel(
      _make_gather_body(h),
      out_type=jax.ShapeDtypeStruct((EP2, 128), f32),
      mesh=mesh,
      scratch_types=(
          [pltpu.VMEM((CPT2, CHUNK), i32)] * 2
          + [pltpu.VMEM((CHUNK, 128), f32)] * 6
          + [pltpu.SemaphoreType.DMA] * 6
      ),
      compiler_params=pltpu.CompilerParams(needs_layout_passes=False),
  )


def _sc_gather(h, *args):
  return _get_sc_gather(h)(*args)


SCHUNK = 64
SNCHUNK = EP // SCHUNK
SNCHUNK2 = SNCHUNK // 2
SCPT2 = SNCHUNK2 // NW


def _make_scatter_body(h):
  def body(s_hbm, o_hbm, sidx, oidx, z64, out, sidx_v, oidx_v,
           buf0, buf1, sem0, sem1, pooled):
    cid = lax.axis_index("c")
    sid = lax.axis_index("s")
    pltpu.sync_copy(z64, buf0)
    for k in range(ROWS_PT // SCHUNK):
      pltpu.sync_copy(buf0, pooled.at[pl.ds(sid * ROWS_PT + k * SCHUNK,
                                            SCHUNK)])
    plsc.subcore_barrier()

    c0l = cid * (SNCHUNK2 // NC) + sid * SCPT2
    c0g = h * SNCHUNK2 + c0l
    pltpu.sync_copy(sidx.at[pl.ds(c0g, SCPT2)], sidx_v)
    pltpu.sync_copy(oidx.at[pl.ds(c0g, SCPT2)], oidx_v)

    bufs = ((buf0, sem0), (buf1, sem1))

    def one_pass(val_hbm, idx_v):
      def issue(j, b, s):
        pltpu.async_copy(val_hbm.at[pl.ds((c0l + j) * SCHUNK, SCHUNK)], b, s)

      def drain_add(j, b, s):
        pltpu.make_async_copy(
            val_hbm.at[pl.ds((c0l + j) * SCHUNK, SCHUNK)], b, s).wait()
        pltpu.sync_copy(b, pooled.at[idx_v.at[j]], add=True)

      issue(0, *bufs[0])
      issue(1, *bufs[1])

      def body_loop(i, carry):
        for p in range(2):
          j = i * 2 + p
          drain_add(j, *bufs[p])
          issue(j + 2, *bufs[p])
        return carry

      lax.fori_loop(0, (SCPT2 - 2) // 2, body_loop, 0)
      for p in range(2):
        drain_add(SCPT2 - 2 + p, *bufs[p])

    one_pass(s_hbm, sidx_v)
    one_pass(o_hbm, oidx_v)
    plsc.subcore_barrier()
    pltpu.sync_copy(pooled.at[pl.ds(sid * ROWS_PT, ROWS_PT)],
                    out.at[pl.ds(cid * NP + sid * ROWS_PT, ROWS_PT)])

  return body


@functools.cache
def _get_sc_scatter(h):
  mesh = plsc.VectorSubcoreMesh(core_axis_name="c", subcore_axis_name="s")
  return pl.kernel(
      _make_scatter_body(h),
      out_type=jax.ShapeDtypeStruct((NC * NP, 128), f32),
      mesh=mesh,
      scratch_types=[
          pltpu.VMEM((SCPT2, SCHUNK), i32),
          pltpu.VMEM((SCPT2, SCHUNK), i32),
          pltpu.VMEM((SCHUNK, 128), f32),
          pltpu.VMEM((SCHUNK, 128), f32),
          pltpu.SemaphoreType.DMA,
          pltpu.SemaphoreType.DMA,
          pltpu.VMEM_SHARED((NP, 128), f32),
      ],
      compiler_params=pltpu.CompilerParams(needs_layout_passes=False),
  )


def _sc_scatter(h, *args):
  return _get_sc_scatter(h)(*args)


def _counts_body(sidx, oidx, znp, out, idx_v, counts_v):
  cid = lax.axis_index("c")
  sid = lax.axis_index("s")
  wid = sid * NC + cid
  pltpu.sync_copy(znp, counts_v)
  ones = jnp.full((16,), 1.0, f32)
  c0 = wid * CPT
  for idx_ref in (sidx, oidx):
    pltpu.sync_copy(idx_ref.at[pl.ds(c0, CPT)], idx_v)

    def body(j, carry):
      for k in range(CHUNK // 16):
        idx16 = idx_v[j, pl.ds(k * 16, 16)]
        plsc.addupdate_scatter(counts_v, [idx16], ones)
      return carry

    lax.fori_loop(0, CPT, body, 0)
  pltpu.sync_copy(counts_v, out.at[wid])


@functools.cache
def _get_sc_counts():
  mesh = plsc.VectorSubcoreMesh(core_axis_name="c", subcore_axis_name="s")
  return pl.kernel(
      _counts_body,
      out_type=jax.ShapeDtypeStruct((NW, NP), f32),
      mesh=mesh,
      scratch_types=[
          pltpu.VMEM((CPT, CHUNK), i32),
          pltpu.VMEM((NP,), f32),
      ],
      compiler_params=pltpu.CompilerParams(needs_layout_passes=False),
  )


def _sc_counts(*args):
  return _get_sc_counts()(*args)


def _cmerge_body(c_ref, out_ref):
  s = jnp.sum(c_ref[...], axis=0, keepdims=True)
  out_ref[...] = jnp.maximum(s, 1.0)


def _tc_counts_merge(cparts):
  return pl.pallas_call(
      _cmerge_body,
      grid=(NP // BN,),
      in_specs=[pl.BlockSpec((NW, BN), lambda i: (0, i))],
      out_specs=pl.BlockSpec((1, BN), lambda i: (0, i)),
      out_shape=jax.ShapeDtypeStruct((1, NP), f32),
  )(cparts)


def _node0_body(objs_ref, boxes_ref, tab_ref, w1s_ref, w1o_ref, p_ref, q_ref):
  oh = (objs_ref[...] == lax.broadcasted_iota(i32, (1, NUM_OBJS), 1)
        ).astype(f32)
  emb = jnp.dot(oh, tab_ref[...], preferred_element_type=f32)
  tvec = jnp.concatenate([emb, boxes_ref[...]], axis=1)
  p_ref[...] = jnp.dot(tvec, w1s_ref[...], preferred_element_type=f32)
  q_ref[...] = jnp.dot(tvec, w1o_ref[...], preferred_element_type=f32)


def _tc_node0(objs2d, boxes_p, tab, w1s, w1o):
  return pl.pallas_call(
      _node0_body,
      grid=(NP // BN,),
      in_specs=[
          pl.BlockSpec((BN, 1), lambda i: (i, 0)),
          pl.BlockSpec((BN, 4), lambda i: (i, 0)),
          pl.BlockSpec(tab.shape, lambda i: (0, 0)),
          pl.BlockSpec(w1s.shape, lambda i: (0, 0)),
          pl.BlockSpec(w1o.shape, lambda i: (0, 0)),
      ],
      out_specs=[pl.BlockSpec((BN, 128), lambda i: (i, 0))] * 2,
      out_shape=[jax.ShapeDtypeStruct((NP, 128), f32)] * 2,
  )(objs2d, boxes_p, tab, w1s, w1o)


def _make_edge_call(din_p, dout, onehot_pred, fuse_rel):
  def body(g_ref, pred_ref, *rest):
    if onehot_pred:
      (ptab_ref, w1p_ref, b1a_ref, w1bs_ref, b1bs_ref, w1bp_ref, b1bp_ref,
       w1bo_ref, b1bo_ref, s_ref, p_ref, o_ref) = rest
      pp = jnp.dot(ptab_ref[...], w1p_ref[...], preferred_element_type=f32)
      oh = (pred_ref[...] == lax.broadcasted_iota(i32, (1, NUM_PREDS), 1)
            ).astype(f32)
      pcon = jnp.dot(oh, pp, preferred_element_type=f32)
    elif fuse_rel:
      (w1p_ref, b1a_ref, w1bs_ref, b1bs_ref, w1bp_ref, b1bp_ref,
       w1bo_ref, b1bo_ref, wr_ref, br_ref, s_ref, p_ref, o_ref) = rest
      pcon = jnp.dot(pred_ref[...], w1p_ref[...], preferred_element_type=f32)
    else:
      (w1p_ref, b1a_ref, w1bs_ref, b1bs_ref, w1bp_ref, b1bp_ref,
       w1bo_ref, b1bo_ref, s_ref, p_ref, o_ref) = rest
      pcon = jnp.dot(pred_ref[...], w1p_ref[...], preferred_element_type=f32)
    h = jax.nn.relu(g_ref[...] + pcon + b1a_ref[...])
    s_ref[...] = jax.nn.relu(
        jnp.dot(h, w1bs_ref[...], preferred_element_type=f32) + b1bs_ref[...])
    newp = jax.nn.relu(
        jnp.dot(h, w1bp_ref[...], preferred_element_type=f32) + b1bp_ref[...])
    o_ref[...] = jax.nn.relu(
        jnp.dot(h, w1bo_ref[...], preferred_element_type=f32) + b1bo_ref[...])
    if fuse_rel:
      p_ref[...] = (jnp.dot(newp, wr_ref[...], preferred_element_type=f32)
                    + br_ref[...])
    else:
      p_ref[...] = newp

  p_width = NUM_PREDS if fuse_rel else dout
  pred_in_w = 1 if onehot_pred else din_p

  def call(g, pred, *weights):
    wspecs = [pl.BlockSpec(w.shape, lambda i: (0, 0)) for w in weights]
    return pl.pallas_call(
        body,
        grid=(EP2 // BE,),
        in_specs=[
            pl.BlockSpec((BE, 128), lambda i: (i, 0)),
            pl.BlockSpec((BE, pred_in_w), lambda i: (i, 0)),
        ] + wspecs,
        out_specs=[
            pl.BlockSpec((BE, 128), lambda i: (i, 0)),
            pl.BlockSpec((BE, p_width), lambda i: (i, 0)),
            pl.BlockSpec((BE, 128), lambda i: (i, 0)),
        ],
        out_shape=[
            jax.ShapeDtypeStruct((EP2, 128), f32),
            jax.ShapeDtypeStruct((EP2, p_width), f32),
            jax.ShapeDtypeStruct((EP2, 128), f32),
        ],
    )(g, pred, *weights)

  return call


def _make_node_call(final):
  if final:
    def body(pa_ref, pb_ref, pc_ref, pd_ref, cnt_ref, w2a_ref, b2a_ref,
             w2b_ref, b2b_ref, wn_ref, bn_ref, a_ref):
      pooled = (pa_ref[...] + pb_ref[...] + pc_ref[...] + pd_ref[...]
                ) / cnt_ref[...]
      h2 = jax.nn.relu(
          jnp.dot(pooled, w2a_ref[...], preferred_element_type=f32)
          + b2a_ref[...])
      obj = jax.nn.relu(
          jnp.dot(h2, w2b_ref[...], preferred_element_type=f32)
          + b2b_ref[...])
      a_ref[...] = (jnp.dot(obj, wn_ref[...], preferred_element_type=f32)
                    + bn_ref[...])
  else:
    def body(pa_ref, pb_ref, pc_ref, pd_ref, cnt_ref, w2a_ref, b2a_ref,
             w2b_ref, b2b_ref, w1s_ref, w1o_ref, a_ref, b_ref):
      pooled = (pa_ref[...] + pb_ref[...] + pc_ref[...] + pd_ref[...]
                ) / cnt_ref[...]
      h2 = jax.nn.relu(
          jnp.dot(pooled, w2a_ref[...], preferred_element_type=f32)
          + b2a_ref[...])
      obj = jax.nn.relu(
          jnp.dot(h2, w2b_ref[...], preferred_element_type=f32)
          + b2b_ref[...])
      a_ref[...] = jnp.dot(obj, w1s_ref[...], preferred_element_type=f32)
      b_ref[...] = jnp.dot(obj, w1o_ref[...], preferred_element_type=f32)

  out_w = NUM_OBJS if final else 128
  n_out = 1 if final else 2

  def call(parts_a, parts_b, cnt, *weights):
    wspecs = [pl.BlockSpec(w.shape, lambda i: (0, 0)) for w in weights]
    out_specs = [pl.BlockSpec((BN, out_w), lambda i: (i, 0))] * n_out
    out_shape = [jax.ShapeDtypeStruct((NP, out_w), f32)] * n_out
    return pl.pallas_call(
        body,
        grid=(NP // BN,),
        in_specs=[
            pl.BlockSpec((BN, 128), lambda i: (i, 0)),
            pl.BlockSpec((BN, 128), lambda i: (i + NP // BN, 0)),
            pl.BlockSpec((BN, 128), lambda i: (i, 0)),
            pl.BlockSpec((BN, 128), lambda i: (i + NP // BN, 0)),
            pl.BlockSpec((BN, 1), lambda i: (i, 0)),
        ] + wspecs,
        out_specs=out_specs[0] if final else out_specs,
        out_shape=out_shape[0] if final else out_shape,
    )(parts_a, parts_a, parts_b, parts_b, cnt, *weights)

  return call


_edge_enc0 = _make_edge_call(EMB, GDIM, onehot_pred=True, fuse_rel=False)
_edge_mid = _make_edge_call(GDIM, GDIM, onehot_pred=False, fuse_rel=False)
_edge_enc4 = _make_edge_call(GDIM, LAT, onehot_pred=False, fuse_rel=False)
_edge_dec = _make_edge_call(LAT, EMB, onehot_pred=False, fuse_rel=True)
_node_mid = _make_node_call(final=False)
_node_final = _make_node_call(final=True)


def _split_w1(p, din_o, din_p, dout):
  w1a, w1b, b1b = p['w1a'], p['w1b'], p['b1b']
  return dict(
      w1s=w1a[:din_o], w1p=w1a[din_o:din_o + din_p], w1o=w1a[din_o + din_p:],
      b1a=p['b1a'][None, :],
      w1bs=w1b[:, :HID], b1bs=b1b[None, :HID],
      w1bp=w1b[:, HID:HID + dout], b1bp=b1b[None, HID:HID + dout],
      w1bo=w1b[:, HID + dout:], b1bo=b1b[None, HID + dout:],
      w2a=p['w2a'], b2a=p['b2a'][None, :],
      w2b=p['w2b'], b2b=p['b2b'][None, :],
  )


_LAYERS = [
    ('enc0', EMB + 4, EMB, GDIM),
    ('enc1', GDIM, GDIM, GDIM),
    ('enc2', GDIM, GDIM, GDIM),
    ('enc3', GDIM, GDIM, GDIM),
    ('enc4', GDIM, GDIM, LAT),
    ('dec', LAT, LAT, EMB),
]


def kernel(objs, edges, predicates, boxes, params):
  s_idx = edges[:, 0].astype(i32)
  o_idx = edges[:, 1].astype(i32)
  pad_e = jnp.full((EP - N_EDGES,), N_NODES, i32)
  s_pad = jnp.concatenate([s_idx, pad_e])
  o_pad = jnp.concatenate([o_idx, pad_e])
  sidx2d = s_pad.reshape(NCHUNK, CHUNK)
  oidx2d = o_pad.reshape(NCHUNK, CHUNK)
  sidx3g = s_pad.reshape(2, NW, CPT2, CHUNK)
  oidx3g = o_pad.reshape(2, NW, CPT2, CHUNK)
  sidx64 = s_pad.reshape(SNCHUNK, SCHUNK)
  oidx64 = o_pad.reshape(SNCHUNK, SCHUNK)
  objs2d = jnp.pad(objs.astype(i32), (0, NP - N_NODES)).reshape(NP, 1)
  boxes_p = jnp.pad(boxes, ((0, NP - N_NODES), (0, 0)))
  preds2d = jnp.pad(predicates.astype(i32), (0, EP - N_EDGES)).reshape(EP, 1)
  z64 = jnp.zeros((SCHUNK, 128), f32)
  znp = jnp.zeros((NP,), f32)

  W = {name: _split_w1(params[name], a, b, c) for name, a, b, c in _LAYERS}

  cparts = _sc_counts(sidx2d, oidx2d, znp)
  cnt = _tc_counts_merge(cparts).reshape(NP, 1)

  P, Q = _tc_node0(objs2d, boxes_p, params['obj_table'][:NUM_OBJS],
                   W['enc0']['w1s'], W['enc0']['w1o'])

  preds_h = (preds2d[:EP2], preds2d[EP2:])
  pred_h = [preds_h[0], preds_h[1]]
  rel_h = [None, None]
  logits_full = None
  for li, (name, din_o, din_p, dout) in enumerate(_LAYERS):
    w = W[name]
    common = (w['b1a'], w['w1bs'], w['b1bs'], w['w1bp'], w['b1bp'],
              w['w1bo'], w['b1bo'])
    S_h = [None, None]
    O_h = [None, None]
    parts_h = [None, None]
    for hh in range(2):
      g = _sc_gather(hh, P, Q, sidx3g[hh], oidx3g[hh])
      if name == 'enc0':
        S, newp, O = _edge_enc0(
            g, pred_h[hh], params['pred_table'], w['w1p'], *common)
      elif name == 'enc4':
        S, newp, O = _edge_enc4(g, pred_h[hh], w['w1p'], *common)
      elif name == 'dec':
        S, newp, O = _edge_dec(
            g, pred_h[hh], w['w1p'], *common,
            params['wr'], params['br'][None, :])
        rel_h[hh] = newp
      else:
        S, newp, O = _edge_mid(g, pred_h[hh], w['w1p'], *common)
      if name != 'dec':
        pred_h[hh] = newp
      S_h[hh], O_h[hh] = S, O
    for hh in range(2):
      parts_h[hh] = _sc_scatter(hh, S_h[hh], O_h[hh], sidx64, oidx64, z64)
    if name == 'dec':
      logits_full = _node_final(parts_h[0], parts_h[1], cnt, w['w2a'],
                                w['b2a'], w['w2b'], w['b2b'], params['wn'],
                                params['bn'][None, :])
    else:
      nxt = _LAYERS[li + 1][0]
      P, Q = _node_mid(parts_h[0], parts_h[1], cnt, w['w2a'], w['b2a'],
                       w['w2b'], w['b2b'], W[nxt]['w1s'], W[nxt]['w1o'])

  rel_full = jnp.concatenate(rel_h, axis=0)
  return logits_full[:N_NODES], rel_full[:N_EDGES]
